# baseline (speedup 1.0000x reference)
import functools

import jax
import jax.numpy as jnp
from jax import lax
from jax.experimental import pallas as pl
from jax.experimental.pallas import tpu as pltpu
from jax.experimental.pallas import tpu_sc as plsc


def _make_sc_gather(B, S, D, N, M):
    info = plsc.get_sparse_core_info()
    NC, NS, L = info.num_cores, info.num_subcores, info.num_lanes
    NW = NC * NS
    RPW = N // NW
    GC = 8

    mesh = plsc.VectorSubcoreMesh(core_axis_name="c", subcore_axis_name="s")

    @functools.partial(
        pl.kernel,
        out_type=jax.ShapeDtypeStruct((B * N, D), jnp.float32),
        mesh=mesh,
        scratch_types=[
            pltpu.VMEM((RPW,), jnp.int32),
            pltpu.VMEM((RPW,), jnp.int32),
            pltpu.VMEM((RPW,), jnp.int32),
            pltpu.VMEM((L,), jnp.int32),
            pltpu.VMEM((RPW,), jnp.int32),
            pltpu.VMEM((RPW, D), jnp.float32),
            pltpu.VMEM((RPW, D), jnp.float32),
            pltpu.SemaphoreType.DMA,
        ],
    )
    def sc_gather(in_hbm, idx_hbm, rk_hbm, gl_hbm, wsc_hbm, mstrows_hbm,
                  out_hbm, idx_v, rank_v, gbase_v, sc_v, gidx_v,
                  stage_v, rows_v, gsem):
        wid = lax.axis_index("s") * NC + lax.axis_index("c")
        base = wid * RPW
        pltpu.sync_copy(idx_hbm.at[pl.ds(base, RPW)], idx_v)
        pltpu.sync_copy(rk_hbm.at[pl.ds(base, RPW)], rank_v)
        pltpu.sync_copy(gl_hbm.at[pl.ds(base, RPW)], gbase_v)
        pltpu.sync_copy(wsc_hbm.at[pl.ds(wid * L, L)], sc_v)
        pltpu.sync_copy(mstrows_hbm, rows_v)

        k_w = sc_v[pl.ds(0, L)][0]

        def per_batch(b, _):
            ivecs, rvecs = [], []
            for c in range(RPW // L):
                sl = pl.ds(c * L, L)
                ivecs.append(idx_v[sl])
                rvecs.append(rank_v[sl])
                gidx_v[sl] = gbase_v[sl] + b * S

            for i in range(RPW // GC):
                @pl.when(i * GC < k_w)
                def _(i=i):
                    pltpu.async_copy(
                        in_hbm.at[gidx_v.at[pl.ds(i * GC, GC)]],
                        stage_v.at[pl.ds(i * GC, GC)], gsem)
            for i in range(RPW // GC):
                @pl.when(i * GC < k_w)
                def _(i=i):
                    pltpu.make_async_copy(
                        in_hbm.at[pl.ds(0, GC)],
                        stage_v.at[pl.ds(i * GC, GC)], gsem).wait()

            for c in range(RPW // L):
                for l in range(L):
                    @pl.when(ivecs[c][l] >= M)
                    def _(c=c, l=l):
                        j = c * L + l
                        r = rvecs[c][l]
                        for k in range(D // L):
                            sl = pl.ds(k * L, L)
                            rows_v[j, sl] = stage_v[r, sl]

            pltpu.sync_copy(rows_v, out_hbm.at[pl.ds(b * N + base, RPW)])
            return 0

        lax.fori_loop(0, B, per_batch, 0)

    return sc_gather


def kernel(inputs, mask_indices, un_masked_indices, mst):
    B, S, D = inputs.shape
    M = mask_indices.shape[0]
    N = M + un_masked_indices.shape[0]
    idx = jnp.concatenate([mask_indices, un_masked_indices]).astype(jnp.int32)

    info = plsc.get_sparse_core_info()
    NW = info.num_cores * info.num_subcores
    RPW = N // NW

    idx2 = idx.reshape(NW, RPW)
    u = (idx2 >= M).astype(jnp.int32)
    k_w = jnp.sum(u, axis=1)
    rank = jnp.cumsum(u, axis=1) - u
    perm = jnp.argsort(1 - u, axis=1, stable=True)
    gl = jnp.take_along_axis(idx2 - M, perm, axis=1)
    cols = jnp.broadcast_to(jnp.arange(RPW, dtype=jnp.int32), (NW, RPW))
    spread = (cols + 7 * jnp.arange(NW, dtype=jnp.int32)[:, None]) * 37 % S
    gl = jnp.where(cols < k_w[:, None], gl, spread).reshape(-1)
    wsc = jnp.zeros((NW, 16), jnp.int32).at[:, 0].set(k_w).reshape(-1)
    mstrows = jnp.broadcast_to(mst.reshape(1, D), (RPW, D)).astype(inputs.dtype)

    sc_gather = _make_sc_gather(B, S, D, N, M)
    out_flat = sc_gather(inputs.reshape(B * S, D), idx, rank.reshape(-1),
                         gl, wsc, mstrows)
    return out_flat.reshape(B, N, D)


# speedup vs baseline: 1.7852x; 1.7852x over previous
import functools

import jax
import jax.numpy as jnp
from jax import lax
from jax.experimental import pallas as pl
from jax.experimental.pallas import tpu as pltpu
from jax.experimental.pallas import tpu_sc as plsc


def _make_sc_gather(B, S, D, N, M):
    info = plsc.get_sparse_core_info()
    NC, NS, L = info.num_cores, info.num_subcores, info.num_lanes
    NW = NC * NS
    RPW = N // NW

    mesh = plsc.VectorSubcoreMesh(core_axis_name="c", subcore_axis_name="s")

    @functools.partial(
        pl.kernel,
        out_type=jax.ShapeDtypeStruct((B * N, D), jnp.float32),
        mesh=mesh,
        scratch_types=[
            pltpu.VMEM((RPW,), jnp.int32),
            pltpu.VMEM((RPW,), jnp.int32),
            pltpu.VMEM((RPW,), jnp.int32),
            pltpu.VMEM((RPW, D), jnp.float32),
            pltpu.VMEM((D,), jnp.float32),
            pltpu.SemaphoreType.DMA,
        ],
    )
    def sc_gather(in_hbm, idx_hbm, gid_hbm, mst_hbm, out_hbm,
                  idx_v, gbase_v, gidx_v, rows_v, mst_v, sem):
        wid = lax.axis_index("s") * NC + lax.axis_index("c")
        base = wid * RPW
        pltpu.sync_copy(idx_hbm.at[pl.ds(base, RPW)], idx_v)
        pltpu.sync_copy(gid_hbm.at[pl.ds(base, RPW)], gbase_v)
        pltpu.sync_copy(mst_hbm, mst_v)

        def per_batch(b, _):
            ivecs = []
            for c in range(RPW // L):
                sl = pl.ds(c * L, L)
                ivecs.append(idx_v[sl])
                gidx_v[sl] = gbase_v[sl] + b * S
            pltpu.async_copy(in_hbm.at[gidx_v], rows_v, sem).wait()

            ms = [mst_v[pl.ds(k * L, L)] for k in range(D // L)]
            for c in range(RPW // L):
                for l in range(L):
                    @pl.when(ivecs[c][l] < M)
                    def _(c=c, l=l):
                        j = c * L + l
                        for k in range(D // L):
                            rows_v[j, pl.ds(k * L, L)] = ms[k]

            pltpu.sync_copy(rows_v, out_hbm.at[pl.ds(b * N + base, RPW)])
            return 0

        lax.fori_loop(0, B, per_batch, 0)

    return sc_gather


def kernel(inputs, mask_indices, un_masked_indices, mst):
    B, S, D = inputs.shape
    M = mask_indices.shape[0]
    N = M + un_masked_indices.shape[0]
    idx = jnp.concatenate([mask_indices, un_masked_indices]).astype(jnp.int32)
    spread = (jnp.arange(N, dtype=jnp.int32) * 37) % S
    gid = jnp.where(idx >= M, idx - M, spread)
    sc_gather = _make_sc_gather(B, S, D, N, M)
    out_flat = sc_gather(inputs.reshape(B * S, D), idx, gid,
                         mst.reshape(D).astype(inputs.dtype))
    return out_flat.reshape(B, N, D)
